# baseline (speedup 1.0000x reference)
import jax
import jax.numpy as jnp
from jax import lax
from jax.experimental import pallas as pl
from jax.experimental.pallas import tpu as pltpu
from jax.experimental.pallas import tpu_sc as plsc

EMBED_DIM = 16
BATCH = 16384
VOCAB_ROWS = 1000000

S1_COLS = 51200
S1_ROWS = S1_COLS // 128
S1_GRID = 20
SCORE_ROWS = S1_GRID * S1_ROWS

NUM_CORES = 2
NUM_SUBCORES = 16
NUM_WORKERS = NUM_CORES * NUM_SUBCORES
BPW = BATCH // NUM_WORKERS
GROUPS = BPW // 16
IDX_TILES = BPW // 128


def _tc_dense_body(w_ref, b_ref, t_ref, o_ref):
    acc = jnp.full((S1_ROWS, 128), b_ref[0], jnp.float32)
    for d in range(EMBED_DIM):
        acc = acc + t_ref[d, :].reshape(S1_ROWS, 128) * w_ref[0, d]
    o_ref[...] = acc


def _sc_gather_body(x_hbm, scores_hbm, out_hbm,
                    idx_v, idx_s, rows_v, out_v, sem):
    wid = lax.axis_index("s") * NUM_CORES + lax.axis_index("c")

    pltpu.sync_copy(x_hbm.at[pl.ds(wid * IDX_TILES, IDX_TILES)], idx_v)
    for t in range(IDX_TILES):
        for u in range(8):
            seg = idx_v[t, pl.ds(u * 16, 16)]
            idx_s[t, pl.ds(u * 16, 16)] = seg >> 7

    copies = [
        pltpu.async_copy(
            scores_hbm.at[idx_s.at[j]],
            rows_v.at[pl.ds(j * 128, 128)], sem)
        for j in range(IDX_TILES)
    ]
    for c in copies:
        c.wait()

    iota = lax.iota(jnp.int32, 16)

    def group(c, carry):
        t = c // 8
        u = c - t * 8
        sub = idx_v[t, pl.ds(u * 16, 16)] & 127
        row_ids = c * 16 + iota
        out_v[pl.ds(c * 16, 16)] = plsc.load_gather(rows_v, [row_ids, sub])
        return carry

    lax.fori_loop(0, GROUPS, group, 0)
    pltpu.sync_copy(out_v, out_hbm.at[pl.ds(wid * BPW, BPW)])


@jax.jit
def kernel(x, table, W, b):
    table_t = table.T
    x32 = x.astype(jnp.int32).reshape(BATCH // 128, 128)

    scores = pl.pallas_call(
        _tc_dense_body,
        grid=(S1_GRID,),
        in_specs=[
            pl.BlockSpec(memory_space=pltpu.SMEM),
            pl.BlockSpec(memory_space=pltpu.SMEM),
            pl.BlockSpec((EMBED_DIM, S1_COLS), lambda k: (0, k)),
        ],
        out_specs=pl.BlockSpec((S1_ROWS, 128), lambda k: (k, 0)),
        out_shape=jax.ShapeDtypeStruct((SCORE_ROWS, 128), jnp.float32),
    )(W, b, table_t)

    mesh = plsc.VectorSubcoreMesh(
        core_axis_name="c", subcore_axis_name="s",
        num_cores=NUM_CORES, num_subcores=NUM_SUBCORES)
    run = pl.kernel(
        _sc_gather_body,
        mesh=mesh,
        out_type=jax.ShapeDtypeStruct((BATCH,), jnp.float32),
        scratch_types=[
            pltpu.VMEM((IDX_TILES, 128), jnp.int32),
            pltpu.VMEM((IDX_TILES, 128), jnp.int32),
            pltpu.VMEM((BPW, 128), jnp.float32),
            pltpu.VMEM((BPW,), jnp.float32),
            pltpu.SemaphoreType.DMA,
        ],
        compiler_params=pltpu.CompilerParams(
            needs_layout_passes=False, use_tc_tiling_on_sc=True),
    )
    return run(x32, scores)


# speedup vs baseline: 1.0980x; 1.0980x over previous
import jax
import jax.numpy as jnp
from jax import lax
from jax.experimental import pallas as pl
from jax.experimental.pallas import tpu as pltpu
from jax.experimental.pallas import tpu_sc as plsc

EMBED_DIM = 16
BATCH = 16384
VOCAB_ROWS = 1000000

S1_COLS = 102400
S1_ROWS = S1_COLS // 128
S1_GRID = 10
SCORE_ROWS = S1_GRID * S1_ROWS

NUM_CORES = 2
NUM_SUBCORES = 16
NUM_WORKERS = NUM_CORES * NUM_SUBCORES
BPW = BATCH // NUM_WORKERS
GROUPS = BPW // 16
IDX_TILES = BPW // 128


def _tc_dense_body(w_ref, b_ref, t_ref, o_ref):
    acc = jnp.full((S1_ROWS, 128), b_ref[0], jnp.float32)
    for d in range(EMBED_DIM):
        acc = acc + t_ref[d, :].reshape(S1_ROWS, 128) * w_ref[0, d]
    o_ref[...] = acc


def _sc_gather_body(x_hbm, scores_hbm, out_hbm,
                    idx_v, idx_s, rows_v, out_v, sem):
    wid = lax.axis_index("s") * NUM_CORES + lax.axis_index("c")

    pltpu.sync_copy(x_hbm.at[pl.ds(wid * IDX_TILES, IDX_TILES)], idx_v)
    for t in range(IDX_TILES):
        for u in range(8):
            seg = idx_v[t, pl.ds(u * 16, 16)]
            idx_s[t, pl.ds(u * 16, 16)] = seg >> 7

    copies = [
        pltpu.async_copy(
            scores_hbm.at[idx_s.at[j]],
            rows_v.at[pl.ds(j * 128, 128)], sem)
        for j in range(IDX_TILES)
    ]
    for c in copies:
        c.wait()

    iota = lax.iota(jnp.int32, 16)

    def group(c, carry):
        t = c // 8
        u = c - t * 8
        sub = idx_v[t, pl.ds(u * 16, 16)] & 127
        row_ids = c * 16 + iota
        out_v[pl.ds(c * 16, 16)] = plsc.load_gather(rows_v, [row_ids, sub])
        return carry

    lax.fori_loop(0, GROUPS, group, 0)
    pltpu.sync_copy(out_v, out_hbm.at[pl.ds(wid * BPW, BPW)])


@jax.jit
def kernel(x, table, W, b):
    table_t = table.T
    x32 = x.astype(jnp.int32).reshape(BATCH // 128, 128)

    scores = pl.pallas_call(
        _tc_dense_body,
        grid=(S1_GRID,),
        in_specs=[
            pl.BlockSpec(memory_space=pltpu.SMEM),
            pl.BlockSpec(memory_space=pltpu.SMEM),
            pl.BlockSpec((EMBED_DIM, S1_COLS), lambda k: (0, k)),
        ],
        out_specs=pl.BlockSpec((S1_ROWS, 128), lambda k: (k, 0)),
        out_shape=jax.ShapeDtypeStruct((SCORE_ROWS, 128), jnp.float32),
    )(W, b, table_t)

    mesh = plsc.VectorSubcoreMesh(
        core_axis_name="c", subcore_axis_name="s",
        num_cores=NUM_CORES, num_subcores=NUM_SUBCORES)
    run = pl.kernel(
        _sc_gather_body,
        mesh=mesh,
        out_type=jax.ShapeDtypeStruct((BATCH,), jnp.float32),
        scratch_types=[
            pltpu.VMEM((IDX_TILES, 128), jnp.int32),
            pltpu.VMEM((IDX_TILES, 128), jnp.int32),
            pltpu.VMEM((BPW, 128), jnp.float32),
            pltpu.VMEM((BPW,), jnp.float32),
            pltpu.SemaphoreType.DMA,
        ],
        compiler_params=pltpu.CompilerParams(
            needs_layout_passes=False, use_tc_tiling_on_sc=True),
    )
    return run(x32, scores)
